# Initial kernel scaffold; baseline (speedup 1.0000x reference)
import jax
import jax.numpy as jnp
from jax.experimental import pallas as pl


def kernel(x, edge_index, edge_attr, params):
    raise NotImplementedError("write your pallas kernel here")


# speedup vs baseline: 1.7572x; 1.7572x over previous
import functools

import jax
import jax.numpy as jnp
from jax import lax
from jax.experimental import pallas as pl
from jax.experimental.pallas import tpu as pltpu

N_NODES = 50000
N_EDGES = 800000
E_TILE = 2048
E_PAD = ((N_EDGES + E_TILE - 1) // E_TILE) * E_TILE


def _edge_u_body(ea_ref, w1t_ref, b1_ref, w2t_ref, c2_ref,
                 o1_ref, o2_ref, o3_ref):
    ea = ea_ref[...]
    outs = (o1_ref, o2_ref, o3_ref)
    for l in range(3):
        t = ea[:, 0:1] * w1t_ref[l, 0:1, :]
        t += ea[:, 1:2] * w1t_ref[l, 1:2, :]
        t += ea[:, 2:3] * w1t_ref[l, 2:3, :]
        t = jnp.maximum(t + b1_ref[l], 0.0)
        u = jnp.dot(t, w2t_ref[l], preferred_element_type=jnp.float32)
        u = u + c2_ref[l]
        outs[l][0] = u[:, :32]
        outs[l][1] = u[:, 32:]


def _edge_u(ea_pad, w1t, b1, w2t, c2):
    grid = (E_PAD // E_TILE,)
    out_sds = jax.ShapeDtypeStruct((2, E_PAD, 32), jnp.float32)
    full = lambda *s: pl.BlockSpec(s, lambda i: tuple(0 for _ in s))
    return pl.pallas_call(
        _edge_u_body,
        grid=grid,
        in_specs=[
            pl.BlockSpec((E_TILE, 3), lambda i: (i, 0)),
            full(3, 3, 64),
            full(3, 1, 64),
            full(3, 64, 64),
            full(3, 1, 64),
        ],
        out_specs=[pl.BlockSpec((2, E_TILE, 32), lambda i: (0, i, 0))] * 3,
        out_shape=[out_sds] * 3,
    )(ea_pad, w1t, b1, w2t, c2)


def _lin(x, w, b):
    return x @ w.T + b


def _mlp(x, w1, b1, w2, b2):
    return _lin(jax.nn.relu(_lin(x, w1, b1)), w2, b2)


def kernel(x, edge_index, edge_attr, params):
    src = edge_index[0]
    dst = edge_index[1]
    sp = params['sp']
    cl = params['cl']
    n = N_NODES

    ea_pad = jnp.pad(edge_attr, ((0, E_PAD - N_EDGES), (0, 0)))
    w1t = jnp.stack([sp['convs'][l]['ew1'].T for l in range(3)])
    b1 = jnp.stack([sp['convs'][l]['eb1'][None, :] for l in range(3)])
    w2t = jnp.stack([(sp['convs'][l]['mw1'][:, 64:] @ sp['convs'][l]['ew2']).T
                     for l in range(3)])
    c2 = jnp.stack([(sp['convs'][l]['eb2'] @ sp['convs'][l]['mw1'][:, 64:].T
                     + sp['convs'][l]['mb1'])[None, :] for l in range(3)])
    u123 = _edge_u(ea_pad, w1t, b1, w2t, c2)
    us = [jnp.concatenate([u[0, :N_EDGES], u[1, :N_EDGES]], axis=-1)
          for u in u123]

    cnt = jnp.zeros((n,), jnp.float32).at[dst].add(1.0)
    cntm = jnp.maximum(cnt, 1.0)
    has = (cnt > 0).astype(jnp.float32)
    dinv = (cnt + 1.0) ** -0.5

    h = _lin(x, sp['inp_w'], sp['inp_b'])
    for l in range(3):
        p = sp['convs'][l]
        xn = _mlp(h, p['nw1'], p['nb1'], p['nw2'], p['nb2'])
        a_tab = xn @ p['mw1'][:, :64].T
        r = jax.nn.relu(a_tab[src] + us[l])
        rsum = jnp.zeros((n, 64), jnp.float32).at[dst].add(r)
        mean = (rsum / cntm[:, None]) @ p['mw2'].T + has[:, None] * p['mb2']
        o = mean + xn
        bn = sp['bns'][l]
        o = o * (bn['g'] / jnp.sqrt(1.0 + 1e-5)) + bn['b']
        h = h + jax.nn.relu(o)
    spatial = _lin(h, sp['out_w'], sp['out_b'])

    x2 = spatial
    for i in range(3):
        g = cl['gcn'][i]
        y = (x2 @ g['w'].T) * dinv[:, None]
        acc = jnp.zeros((n, 64), jnp.float32).at[dst].add(y[src])
        xn2 = jax.nn.relu(dinv[:, None] * (acc + y) + g['b'])
        x2 = x2 + xn2 if i > 0 else xn2

    att = jax.nn.sigmoid(_lin(jax.nn.relu(_lin(x2, cl['att_w1'], cl['att_b1'])),
                              cl['att_w2'], cl['att_b2']))
    x2 = x2 * att
    logits = _lin(jax.nn.relu(_lin(x2, cl['cls_w1'], cl['cls_b1'])),
                  cl['cls_w2'], cl['cls_b2'])
    return spatial, logits
